# Initial kernel scaffold; baseline (speedup 1.0000x reference)
import jax
import jax.numpy as jnp
from jax.experimental import pallas as pl


def kernel(ids, word_embedding):
    raise NotImplementedError("write your pallas kernel here")


# speedup vs baseline: 1.1001x; 1.1001x over previous
import functools

import numpy as np
import jax
import jax.numpy as jnp
from jax import lax
from jax.experimental import pallas as pl
from jax.experimental.pallas import tpu as pltpu
from jax.experimental.pallas import tpu_sc as plsc

VOCAB = 100000
EMBD = 128
MAX_LEN = 8192
BATCH = 4
SEQ = 8192
TOK = BATCH * SEQ
NW = 32
PER = TOK // NW
CHUNK = 128
NCHUNK = PER // CHUNK
LANES = 16


def _position_table() -> np.ndarray:
    pos = np.arange(MAX_LEN, dtype=np.float64)[:, None]
    div = np.arange(0, EMBD, 2, dtype=np.float64)[None, :]
    m = (pos / (10000.0 ** (div / EMBD))).astype(np.float32)
    return np.concatenate([np.sin(m), np.cos(m)], axis=-1).astype(np.float32)


_POS = _position_table()

_MESH = plsc.VectorSubcoreMesh(core_axis_name="c", subcore_axis_name="s")


@functools.partial(
    pl.kernel,
    mesh=_MESH,
    out_type=[
        jax.ShapeDtypeStruct((TOK, EMBD), jnp.float32),
        jax.ShapeDtypeStruct((NW, LANES), jnp.int32),
    ],
    scratch_types=[
        pltpu.VMEM((PER,), jnp.int32),
        pltpu.VMEM((CHUNK, EMBD), jnp.float32),
        pltpu.VMEM((CHUNK, EMBD), jnp.float32),
        pltpu.VMEM((LANES,), jnp.int32),
        pltpu.SemaphoreType.DMA,
        pltpu.SemaphoreType.DMA,
    ],
)
def _embed_sc(ids_h, tab_h, pos_h, out_h, cnt_h,
              idx_v, gbuf, pbuf, cnt_v, gsem, psem):
    wid = lax.axis_index("s") * 2 + lax.axis_index("c")
    base = wid * PER
    sbase = (wid % (SEQ // PER)) * PER

    pltpu.sync_copy(ids_h.at[pl.ds(base, PER)], idx_v)

    def count_body(t, acc):
        v = idx_v[pl.ds(t * LANES, LANES)]
        return acc + jnp.where(v == 1, 1, 0).astype(jnp.int32)

    acc = lax.fori_loop(0, PER // LANES, count_body,
                        jnp.zeros((LANES,), jnp.int32))
    cnt_v[...] = acc
    pltpu.sync_copy(cnt_v, cnt_h.at[wid])

    for c in range(NCHUNK):
        gcp = pltpu.async_copy(tab_h.at[idx_v.at[pl.ds(c * CHUNK, CHUNK)]],
                               gbuf, gsem)
        pcp = pltpu.async_copy(pos_h.at[pl.ds(sbase + c * CHUNK, CHUNK)],
                               pbuf, psem)
        gcp.wait()
        pcp.wait()

        def add_body(r, _):
            for j in range(EMBD // LANES):
                vec = pbuf[r, pl.ds(j * LANES, LANES)]
                plsc.addupdate(gbuf.at[r, pl.ds(j * LANES, LANES)], vec)
            return 0

        lax.fori_loop(0, CHUNK, add_body, 0)
        pltpu.sync_copy(gbuf, out_h.at[pl.ds(base + c * CHUNK, CHUNK)])


def kernel(ids, word_embedding):
    ids_flat = ids.reshape(TOK)
    pos = jnp.asarray(_POS)
    out_flat, cnt = _embed_sc(ids_flat, word_embedding, pos)
    out = out_flat.reshape(BATCH, SEQ, EMBD)
    padding_len = cnt.reshape(BATCH, NW // BATCH, LANES).sum(axis=(1, 2))
    return (out, padding_len)


# speedup vs baseline: 1.3688x; 1.2442x over previous
import functools

import numpy as np
import jax
import jax.numpy as jnp
from jax import lax
from jax.experimental import pallas as pl
from jax.experimental.pallas import tpu as pltpu
from jax.experimental.pallas import tpu_sc as plsc

VOCAB = 100000
EMBD = 128
MAX_LEN = 8192
BATCH = 4
SEQ = 8192
TOK = BATCH * SEQ
NW = 32
PER = TOK // NW
CHUNK = 128
NCHUNK = PER // CHUNK
LANES = 16


def _position_table() -> np.ndarray:
    pos = np.arange(MAX_LEN, dtype=np.float64)[:, None]
    div = np.arange(0, EMBD, 2, dtype=np.float64)[None, :]
    m = (pos / (10000.0 ** (div / EMBD))).astype(np.float32)
    return np.concatenate([np.sin(m), np.cos(m)], axis=-1).astype(np.float32)


_POS = _position_table()

_MESH = plsc.VectorSubcoreMesh(core_axis_name="c", subcore_axis_name="s")


@functools.partial(
    pl.kernel,
    mesh=_MESH,
    out_type=[
        jax.ShapeDtypeStruct((TOK, EMBD), jnp.float32),
        jax.ShapeDtypeStruct((NW, LANES), jnp.int32),
    ],
    scratch_types=[
        pltpu.VMEM((PER,), jnp.int32),
        pltpu.VMEM((CHUNK, EMBD), jnp.float32),
        pltpu.VMEM((CHUNK, EMBD), jnp.float32),
        pltpu.VMEM((CHUNK, EMBD), jnp.float32),
        pltpu.VMEM((CHUNK, EMBD), jnp.float32),
        pltpu.VMEM((LANES,), jnp.int32),
        pltpu.SemaphoreType.DMA,
        pltpu.SemaphoreType.DMA,
        pltpu.SemaphoreType.DMA,
        pltpu.SemaphoreType.DMA,
        pltpu.SemaphoreType.DMA,
        pltpu.SemaphoreType.DMA,
    ],
)
def _embed_sc(ids_h, tab_h, pos_h, out_h, cnt_h,
              idx_v, gbuf0, gbuf1, pbuf0, pbuf1, cnt_v,
              gsem0, gsem1, psem0, psem1, osem0, osem1):
    wid = lax.axis_index("s") * 2 + lax.axis_index("c")
    base = wid * PER
    sbase = (wid % (SEQ // PER)) * PER

    gbufs, pbufs = (gbuf0, gbuf1), (pbuf0, pbuf1)
    gsems, psems, osems = (gsem0, gsem1), (psem0, psem1), (osem0, osem1)

    pltpu.sync_copy(ids_h.at[pl.ds(base, PER)], idx_v)

    def issue(c):
        b = c % 2
        g = pltpu.async_copy(tab_h.at[idx_v.at[pl.ds(c * CHUNK, CHUNK)]],
                             gbufs[b], gsems[b])
        p = pltpu.async_copy(pos_h.at[pl.ds(sbase + c * CHUNK, CHUNK)],
                             pbufs[b], psems[b])
        return g, p

    inflight = [None] * NCHUNK
    ostores = [None] * NCHUNK
    inflight[0] = issue(0)

    def count_body(t, acc):
        v = idx_v[pl.ds(t * LANES, LANES)]
        return acc + jnp.where(v == 1, 1, 0).astype(jnp.int32)

    acc = lax.fori_loop(0, PER // LANES, count_body,
                        jnp.zeros((LANES,), jnp.int32))
    cnt_v[...] = acc
    pltpu.sync_copy(cnt_v, cnt_h.at[wid])

    for c in range(NCHUNK):
        b = c % 2
        if c + 1 < NCHUNK:
            if c >= 1:
                ostores[c - 1].wait()
            inflight[c + 1] = issue(c + 1)
        g, p = inflight[c]
        g.wait()
        p.wait()
        gbuf, pbuf = gbufs[b], pbufs[b]

        def add_body(r, _):
            for j in range(EMBD // LANES):
                vec = pbuf[r, pl.ds(j * LANES, LANES)]
                plsc.addupdate(gbuf.at[r, pl.ds(j * LANES, LANES)], vec)
            return 0

        lax.fori_loop(0, CHUNK, add_body, 0)
        ostores[c] = pltpu.async_copy(
            gbuf, out_h.at[pl.ds(base + c * CHUNK, CHUNK)], osems[b])
    ostores[NCHUNK - 2].wait()
    ostores[NCHUNK - 1].wait()


def kernel(ids, word_embedding):
    ids_flat = ids.reshape(TOK)
    pos = jnp.asarray(_POS)
    out_flat, cnt = _embed_sc(ids_flat, word_embedding, pos)
    out = out_flat.reshape(BATCH, SEQ, EMBD)
    padding_len = cnt.reshape(BATCH, NW // BATCH, LANES).sum(axis=(1, 2))
    return (out, padding_len)


# speedup vs baseline: 1.4783x; 1.0800x over previous
import functools

import numpy as np
import jax
import jax.numpy as jnp
from jax import lax
from jax.experimental import pallas as pl
from jax.experimental.pallas import tpu as pltpu
from jax.experimental.pallas import tpu_sc as plsc

VOCAB = 100000
EMBD = 128
MAX_LEN = 8192
BATCH = 4
SEQ = 8192
TOK = BATCH * SEQ
NW = 32
SRANGE = SEQ // NW
PER = BATCH * SRANGE
CHUNK = 128
NCHUNK = PER // CHUNK
HALVES = SRANGE // CHUNK
LANES = 16


def _position_table() -> np.ndarray:
    pos = np.arange(MAX_LEN, dtype=np.float64)[:, None]
    div = np.arange(0, EMBD, 2, dtype=np.float64)[None, :]
    m = (pos / (10000.0 ** (div / EMBD))).astype(np.float32)
    return np.concatenate([np.sin(m), np.cos(m)], axis=-1).astype(np.float32)


_POS = _position_table()

_MESH = plsc.VectorSubcoreMesh(core_axis_name="c", subcore_axis_name="s")


@functools.partial(
    pl.kernel,
    mesh=_MESH,
    out_type=[
        jax.ShapeDtypeStruct((TOK, EMBD), jnp.float32),
        jax.ShapeDtypeStruct((NW, BATCH * LANES), jnp.int32),
    ],
    scratch_types=[
        pltpu.VMEM((PER,), jnp.int32),
        pltpu.VMEM((SRANGE, EMBD), jnp.float32),
        pltpu.VMEM((CHUNK, EMBD), jnp.float32),
        pltpu.VMEM((CHUNK, EMBD), jnp.float32),
        pltpu.VMEM((CHUNK, EMBD), jnp.float32),
        pltpu.VMEM((BATCH * LANES,), jnp.int32),
        pltpu.SemaphoreType.DMA,
        pltpu.SemaphoreType.DMA,
        pltpu.SemaphoreType.DMA,
        pltpu.SemaphoreType.DMA,
        pltpu.SemaphoreType.DMA,
        pltpu.SemaphoreType.DMA,
    ],
)
def _embed_sc(ids_h, tab_h, pos_h, out_h, cnt_h,
              idx_v, pbuf, gbuf0, gbuf1, gbuf2, cnt_v,
              psem, gsem0, gsem1, gsem2, osem0, osem1):
    wid = lax.axis_index("s") * 2 + lax.axis_index("c")
    sbase = wid * SRANGE

    pcp = pltpu.async_copy(pos_h.at[pl.ds(sbase, SRANGE)], pbuf, psem)

    for b in range(BATCH):
        pltpu.sync_copy(ids_h.at[pl.ds(b * SEQ + sbase, SRANGE)],
                        idx_v.at[pl.ds(b * SRANGE, SRANGE)])

    gbufs = (gbuf0, gbuf1, gbuf2)
    gsems = (gsem0, gsem1, gsem2)
    osems = (osem0, osem1)

    def issue(c):
        return pltpu.async_copy(
            tab_h.at[idx_v.at[pl.ds(c * CHUNK, CHUNK)]],
            gbufs[c % 3], gsems[c % 3])

    inflight = [None] * NCHUNK
    ostores = [None] * NCHUNK
    inflight[0] = issue(0)
    inflight[1] = issue(1)

    for b in range(BATCH):
        def count_body(t, acc):
            v = idx_v[pl.ds(b * SRANGE + t * LANES, LANES)]
            return acc + jnp.where(v == 1, 1, 0).astype(jnp.int32)

        acc = lax.fori_loop(0, SRANGE // LANES, count_body,
                            jnp.zeros((LANES,), jnp.int32))
        cnt_v[pl.ds(b * LANES, LANES)] = acc
    pltpu.sync_copy(cnt_v, cnt_h.at[wid])

    pcp.wait()
    for c in range(NCHUNK):
        if c + 2 < NCHUNK:
            if c >= 1:
                ostores[c - 1].wait()
            inflight[c + 2] = issue(c + 2)
        inflight[c].wait()
        gbuf = gbufs[c % 3]
        poff = (c % HALVES) * CHUNK

        def add_body(r, _):
            for j in range(EMBD // LANES):
                vec = pbuf[poff + r, pl.ds(j * LANES, LANES)]
                plsc.addupdate(gbuf.at[r, pl.ds(j * LANES, LANES)], vec)
            return 0

        lax.fori_loop(0, CHUNK, add_body, 0)
        obase = (c // HALVES) * SEQ + sbase + poff
        ostores[c] = pltpu.async_copy(
            gbuf, out_h.at[pl.ds(obase, CHUNK)], osems[c % 2])
    ostores[NCHUNK - 3].wait()
    ostores[NCHUNK - 2].wait()
    ostores[NCHUNK - 1].wait()


def kernel(ids, word_embedding):
    ids_flat = ids.reshape(TOK)
    pos = jnp.asarray(_POS)
    out_flat, cnt = _embed_sc(ids_flat, word_embedding, pos)
    out = out_flat.reshape(BATCH, SEQ, EMBD)
    padding_len = cnt.reshape(NW, BATCH, LANES).sum(axis=(0, 2))
    return (out, padding_len)


# speedup vs baseline: 1.5205x; 1.0286x over previous
import functools

import numpy as np
import jax
import jax.numpy as jnp
from jax import lax
from jax.experimental import pallas as pl
from jax.experimental.pallas import tpu as pltpu
from jax.experimental.pallas import tpu_sc as plsc

VOCAB = 100000
EMBD = 128
MAX_LEN = 8192
BATCH = 4
SEQ = 8192
TOK = BATCH * SEQ
NW = 32
SRANGE = SEQ // NW
PER = BATCH * SRANGE
CHUNK = 128
NCHUNK = PER // CHUNK
HALVES = SRANGE // CHUNK
LANES = 16


def _position_table() -> np.ndarray:
    pos = np.arange(MAX_LEN, dtype=np.float64)[:, None]
    div = np.arange(0, EMBD, 2, dtype=np.float64)[None, :]
    m = (pos / (10000.0 ** (div / EMBD))).astype(np.float32)
    return np.concatenate([np.sin(m), np.cos(m)], axis=-1).astype(np.float32)


_POS_FLAT = _position_table().reshape(-1)

_MESH = plsc.VectorSubcoreMesh(core_axis_name="c", subcore_axis_name="s")


@functools.partial(
    pl.kernel,
    mesh=_MESH,
    out_type=[
        jax.ShapeDtypeStruct((BATCH, SEQ, EMBD), jnp.float32),
        jax.ShapeDtypeStruct((NW, BATCH * LANES), jnp.int32),
    ],
    scratch_types=[
        pltpu.VMEM((PER,), jnp.int32),
        pltpu.VMEM((SRANGE * EMBD,), jnp.float32),
        pltpu.VMEM((CHUNK, EMBD), jnp.float32),
        pltpu.VMEM((CHUNK, EMBD), jnp.float32),
        pltpu.VMEM((CHUNK, EMBD), jnp.float32),
        pltpu.VMEM((BATCH * LANES,), jnp.int32),
        pltpu.SemaphoreType.DMA,
        pltpu.SemaphoreType.DMA,
        pltpu.SemaphoreType.DMA,
        pltpu.SemaphoreType.DMA,
        pltpu.SemaphoreType.DMA,
        pltpu.SemaphoreType.DMA,
    ],
)
def _embed_sc(ids_h, tab_h, pos_h, out_h, cnt_h,
              idx_v, pbuf, gbuf0, gbuf1, gbuf2, cnt_v,
              psem, gsem0, gsem1, gsem2, osem0, osem1):
    wid = lax.axis_index("s") * 2 + lax.axis_index("c")
    sbase = wid * SRANGE

    pcp = pltpu.async_copy(pos_h.at[pl.ds(sbase * EMBD, SRANGE * EMBD)],
                           pbuf, psem)

    for b in range(BATCH):
        pltpu.sync_copy(ids_h.at[b, pl.ds(sbase, SRANGE)],
                        idx_v.at[pl.ds(b * SRANGE, SRANGE)])

    gbufs = (gbuf0, gbuf1, gbuf2)
    gsems = (gsem0, gsem1, gsem2)
    osems = (osem0, osem1)

    def issue(c):
        return pltpu.async_copy(
            tab_h.at[idx_v.at[pl.ds(c * CHUNK, CHUNK)]],
            gbufs[c % 3], gsems[c % 3])

    inflight = [None] * NCHUNK
    ostores = [None] * NCHUNK
    inflight[0] = issue(0)
    inflight[1] = issue(1)

    for b in range(BATCH):
        def count_body(t, acc):
            v = idx_v[pl.ds(b * SRANGE + t * LANES, LANES)]
            return acc + jnp.where(v == 1, 1, 0).astype(jnp.int32)

        acc = lax.fori_loop(0, SRANGE // LANES, count_body,
                            jnp.zeros((LANES,), jnp.int32))
        cnt_v[pl.ds(b * LANES, LANES)] = acc
    pltpu.sync_copy(cnt_v, cnt_h.at[wid])

    pcp.wait()
    for c in range(NCHUNK):
        if c + 2 < NCHUNK:
            if c >= 1:
                ostores[c - 1].wait()
            inflight[c + 2] = issue(c + 2)
        inflight[c].wait()
        gbuf = gbufs[c % 3]
        poff = (c % HALVES) * CHUNK

        def add_body(r, _):
            pb = (poff + r) * EMBD
            for j in range(EMBD // LANES):
                vec = pbuf[pl.ds(pb + j * LANES, LANES)]
                plsc.addupdate(gbuf.at[r, pl.ds(j * LANES, LANES)], vec)
            return 0

        lax.fori_loop(0, CHUNK, add_body, 0)
        ostores[c] = pltpu.async_copy(
            gbuf, out_h.at[c // HALVES, pl.ds(sbase + poff, CHUNK)],
            osems[c % 2])
    ostores[NCHUNK - 3].wait()
    ostores[NCHUNK - 2].wait()
    ostores[NCHUNK - 1].wait()


def kernel(ids, word_embedding):
    pos = jnp.asarray(_POS_FLAT)
    out, cnt = _embed_sc(ids, word_embedding, pos)
    padding_len = cnt.reshape(NW, BATCH, LANES).sum(axis=(0, 2))
    return (out, padding_len)
